# Initial kernel scaffold; baseline (speedup 1.0000x reference)
import jax
import jax.numpy as jnp
from jax.experimental import pallas as pl


def kernel(token_ids, codebooks, codes):
    raise NotImplementedError("write your pallas kernel here")


# speedup vs baseline: 30.0269x; 30.0269x over previous
import functools

import jax
import jax.numpy as jnp
from jax import lax
from jax.experimental import pallas as pl
from jax.experimental.pallas import tpu as pltpu
from jax.experimental.pallas import tpu_sc as plsc

N_CB = 8
CB_ROWS = 2048
SUB = 16
NC = 2
NS = 16
NW = NC * NS
CHUNK = 512
IDX_W = 128


@functools.lru_cache(maxsize=None)
def _build(n_tok: int, vocab: int):
  assert n_tok % (NW * CHUNK) == 0
  per_w = n_tok // NW
  n_chunks = per_w // CHUNK
  n_idx = CHUNK * N_CB

  mesh = plsc.VectorSubcoreMesh(core_axis_name="c", subcore_axis_name="s")

  @functools.partial(
      pl.kernel,
      out_type=jax.ShapeDtypeStruct((n_tok * N_CB, SUB), jnp.float32),
      mesh=mesh,
      scratch_types=[
          pltpu.VMEM_SHARED((CB_ROWS, SUB), jnp.float32),
          pltpu.VMEM((CHUNK,), jnp.int32),
          pltpu.VMEM((CHUNK,), jnp.int32),
          pltpu.VMEM((CHUNK, 2 * N_CB), jnp.int32),
          pltpu.VMEM((n_idx,), jnp.int32),
          pltpu.VMEM((n_idx, SUB), jnp.float32),
          pltpu.SemaphoreType.DMA,
          pltpu.SemaphoreType.DMA,
      ],
      compiler_params=pltpu.CompilerParams(
          needs_layout_passes=False, use_tc_tiling_on_sc=False
      ),
  )
  def pq(tok_hbm, cb_hbm, codes16_hbm, out_hbm,
         cb_sh, tok_v, ptok_v, codes_v, idx2_v, out_v, sem_g, sem_o):
    cid = lax.axis_index("c")
    sid = lax.axis_index("s")
    wid = cid * NS + sid

    @pl.when(sid == 0)
    def _():
      pltpu.sync_copy(cb_hbm, cb_sh)
    plsc.subcore_barrier()

    lane = lax.iota(jnp.int32, 16)
    sub_lane = lane & 7
    pattern = sub_lane << 8
    low_half = lane < 8
    pair_sel = lane >> 3

    base0 = wid * per_w

    def chunk_body(ck, carry):
      base = base0 + ck * CHUNK
      pltpu.sync_copy(tok_hbm.at[pl.ds(base, CHUNK)], tok_v)

      def mk_ptok(j, c):
        ptok_v[pl.ds(j * 16, 16)] = tok_v[pl.ds(j * 16, 16)] >> 1
        return c
      lax.fori_loop(0, CHUNK // 16, mk_ptok, 0, unroll=8)

      hs = []
      for j in range(CHUNK // IDX_W):
        hs.append(pltpu.async_copy(
            codes16_hbm.at[ptok_v.at[pl.ds(j * IDX_W, IDX_W)]],
            codes_v.at[pl.ds(j * IDX_W, IDX_W)],
            sem_g,
        ))
      for h in hs:
        h.wait()

      def ew(k, c):
        va = codes_v[2 * k]
        vb = codes_v[2 * k + 1]
        pv = plsc.load_gather(tok_v, [2 * k + pair_sel])
        sel = ((pv & 1) << 3) + sub_lane
        ga = jnp.take_along_axis(va, sel, axis=0, mode="promise_in_bounds")
        gb = jnp.take_along_axis(vb, sel, axis=0, mode="promise_in_bounds")
        idx2_v[pl.ds(k * 16, 16)] = jnp.where(low_half, ga, gb) + pattern
        return c
      lax.fori_loop(0, n_idx // 16, ew, 0, unroll=8)

      def gather_group(g, c):
        hs2 = []
        for j in range(8):
          off = g * (8 * IDX_W) + j * IDX_W
          hs2.append(pltpu.async_copy(
              cb_sh.at[idx2_v.at[pl.ds(off, IDX_W)]],
              out_v.at[pl.ds(off, IDX_W)],
              sem_o,
          ))
        for h in hs2:
          h.wait()
        return c
      lax.fori_loop(0, n_idx // (8 * IDX_W), gather_group, 0)

      pltpu.sync_copy(out_v, out_hbm.at[pl.ds(base * N_CB, n_idx)])
      return carry

    lax.fori_loop(0, n_chunks, chunk_body, 0)

  return pq


def kernel(token_ids, codebooks, codes):
  b, l = token_ids.shape
  n_cb, n_codes, sub = codebooks.shape
  tok = token_ids.reshape(-1).astype(jnp.int32)
  cb = codebooks.reshape(n_cb * n_codes, sub).astype(jnp.float32)
  codes16 = codes.astype(jnp.int32).reshape(codes.shape[0] // 2, 2 * n_cb)
  out = _build(b * l, codes.shape[0])(tok, cb, codes16)
  return out.reshape(b, l, n_cb * sub)


# speedup vs baseline: 38.0179x; 1.2661x over previous
import functools

import jax
import jax.numpy as jnp
from jax import lax
from jax.experimental import pallas as pl
from jax.experimental.pallas import tpu as pltpu
from jax.experimental.pallas import tpu_sc as plsc

N_CB = 8
CB_ROWS = 2048
SUB = 16
NC = 2
NS = 16
NW = NC * NS
CHUNK = 256
IDX_W = 128


@functools.lru_cache(maxsize=None)
def _build(n_tok: int, vocab: int):
  assert n_tok % (NW * CHUNK) == 0
  per_w = n_tok // NW
  n_chunks = per_w // CHUNK
  n_idx = CHUNK * N_CB

  mesh = plsc.VectorSubcoreMesh(core_axis_name="c", subcore_axis_name="s")

  @functools.partial(
      pl.kernel,
      out_type=jax.ShapeDtypeStruct((n_tok * N_CB, SUB), jnp.float32),
      mesh=mesh,
      scratch_types=[
          pltpu.VMEM_SHARED((CB_ROWS, SUB), jnp.float32),
          pltpu.VMEM((2, CHUNK), jnp.int32),
          pltpu.VMEM((2, CHUNK), jnp.int32),
          pltpu.VMEM((2, CHUNK, 2 * N_CB), jnp.int32),
          pltpu.VMEM((n_idx,), jnp.int32),
          pltpu.VMEM((2, n_idx, SUB), jnp.float32),
          pltpu.SemaphoreType.DMA,
          pltpu.SemaphoreType.DMA,
          pltpu.SemaphoreType.DMA,
      ],
      compiler_params=pltpu.CompilerParams(
          needs_layout_passes=False, use_tc_tiling_on_sc=False
      ),
  )
  def pq(tok_hbm, cb_hbm, codes16_hbm, out_hbm,
         cb_sh, tok_v, ptok_v, codes_v, idx2_v, out_v, sem_g, sem_o, sem_w):
    cid = lax.axis_index("c")
    sid = lax.axis_index("s")
    wid = cid * NS + sid

    @pl.when(sid == 0)
    def _():
      pltpu.sync_copy(cb_hbm, cb_sh)
    plsc.subcore_barrier()

    lane = lax.iota(jnp.int32, 16)
    sub_lane = lane & 7
    pattern = sub_lane << 8
    low_half = lane < 8
    pair_sel = lane >> 3

    base0 = wid * per_w

    def fetch(g, slot):
      base = base0 + g * CHUNK
      pltpu.sync_copy(tok_hbm.at[pl.ds(base, CHUNK)], tok_v.at[slot])

      def mk_ptok(j, c):
        ptok_v[slot, pl.ds(j * 16, 16)] = tok_v[slot, pl.ds(j * 16, 16)] >> 1
        return c
      lax.fori_loop(0, CHUNK // 16, mk_ptok, 0, unroll=8)

      for j in range(CHUNK // IDX_W):
        pltpu.async_copy(
            codes16_hbm.at[ptok_v.at[slot, pl.ds(j * IDX_W, IDX_W)]],
            codes_v.at[slot, pl.ds(j * IDX_W, IDX_W)],
            sem_g,
        )

    def wait_codes(g, slot):
      for j in range(CHUNK // IDX_W):
        pltpu.make_async_copy(
            codes16_hbm.at[ptok_v.at[slot, pl.ds(j * IDX_W, IDX_W)]],
            codes_v.at[slot, pl.ds(j * IDX_W, IDX_W)],
            sem_g,
        ).wait()

    def out_copy_descriptor(g, slot):
      base = base0 + g * CHUNK
      return pltpu.make_async_copy(
          out_v.at[slot],
          out_hbm.at[pl.ds(base * N_CB, n_idx)],
          sem_w,
      )

    fetch(0, 0)

    def chunk_body(g, carry):
      p = g & 1
      q = 1 - p
      wait_codes(g, p)

      def ew(k, c):
        va = codes_v[p, 2 * k]
        vb = codes_v[p, 2 * k + 1]
        pv = plsc.load_gather(tok_v.at[p], [2 * k + pair_sel])
        sel = ((pv & 1) << 3) + sub_lane
        ga = jnp.take_along_axis(va, sel, axis=0, mode="promise_in_bounds")
        gb = jnp.take_along_axis(vb, sel, axis=0, mode="promise_in_bounds")
        idx2_v[pl.ds(k * 16, 16)] = jnp.where(low_half, ga, gb) + pattern
        return c
      lax.fori_loop(0, n_idx // 16, ew, 0, unroll=8)

      @pl.when(g >= 2)
      def _():
        out_copy_descriptor(g - 2, p).wait()

      for j in range(n_idx // IDX_W):
        pltpu.async_copy(
            cb_sh.at[idx2_v.at[pl.ds(j * IDX_W, IDX_W)]],
            out_v.at[p, pl.ds(j * IDX_W, IDX_W)],
            sem_o,
        )

      @pl.when(g + 1 < n_chunks)
      def _():
        fetch(g + 1, q)

      for j in range(n_idx // IDX_W):
        pltpu.make_async_copy(
            cb_sh.at[idx2_v.at[pl.ds(j * IDX_W, IDX_W)]],
            out_v.at[p, pl.ds(j * IDX_W, IDX_W)],
            sem_o,
        ).wait()
      out_copy_descriptor(g, p).start()
      return carry

    lax.fori_loop(0, n_chunks, chunk_body, 0)

    out_copy_descriptor(n_chunks - 2, (n_chunks - 2) & 1).wait()
    out_copy_descriptor(n_chunks - 1, (n_chunks - 1) & 1).wait()

  return pq


def kernel(token_ids, codebooks, codes):
  b, l = token_ids.shape
  n_cb, n_codes, sub = codebooks.shape
  tok = token_ids.reshape(-1).astype(jnp.int32)
  cb = codebooks.reshape(n_cb * n_codes, sub).astype(jnp.float32)
  codes16 = codes.astype(jnp.int32).reshape(codes.shape[0] // 2, 2 * n_cb)
  out = _build(b * l, codes.shape[0])(tok, cb, codes16)
  return out.reshape(b, l, n_cb * sub)


# speedup vs baseline: 38.6540x; 1.0167x over previous
import functools

import jax
import jax.numpy as jnp
from jax import lax
from jax.experimental import pallas as pl
from jax.experimental.pallas import tpu as pltpu
from jax.experimental.pallas import tpu_sc as plsc

N_CB = 8
CB_ROWS = 2048
SUB = 16
NC = 2
NS = 16
NW = NC * NS
CHUNK = 256
IDX_W = 128


@functools.lru_cache(maxsize=None)
def _build(n_tok: int, vocab: int):
  assert n_tok % (NW * CHUNK) == 0
  per_w = n_tok // NW
  n_chunks = per_w // CHUNK
  n_idx = CHUNK * N_CB

  mesh = plsc.VectorSubcoreMesh(core_axis_name="c", subcore_axis_name="s")

  @functools.partial(
      pl.kernel,
      out_type=jax.ShapeDtypeStruct((n_tok * N_CB, SUB), jnp.float32),
      mesh=mesh,
      scratch_types=[
          pltpu.VMEM_SHARED((CB_ROWS, SUB), jnp.float32),
          pltpu.VMEM((2, CHUNK), jnp.int32),
          pltpu.VMEM((2, CHUNK, N_CB), jnp.int32),
          pltpu.VMEM((n_idx,), jnp.int32),
          pltpu.VMEM((2, n_idx, SUB), jnp.float32),
          pltpu.SemaphoreType.DMA,
          pltpu.SemaphoreType.DMA,
          pltpu.SemaphoreType.DMA,
      ],
      compiler_params=pltpu.CompilerParams(
          needs_layout_passes=False, use_tc_tiling_on_sc=False
      ),
  )
  def pq(tok_hbm, cb_hbm, codes_hbm, out_hbm,
         cb_sh, tok_v, codes_v, idx2_v, out_v, sem_g, sem_o, sem_w):
    cid = lax.axis_index("c")
    sid = lax.axis_index("s")
    wid = cid * NS + sid

    @pl.when(sid == 0)
    def _():
      pltpu.sync_copy(cb_hbm, cb_sh)
    plsc.subcore_barrier()

    lane = lax.iota(jnp.int32, 16)
    row_off = lane >> 3
    col_idx = lane & 7
    pattern = col_idx << 8

    base0 = wid * per_w

    def fetch(g, slot):
      base = base0 + g * CHUNK
      pltpu.sync_copy(tok_hbm.at[pl.ds(base, CHUNK)], tok_v.at[slot])
      for j in range(CHUNK // IDX_W):
        pltpu.async_copy(
            codes_hbm.at[tok_v.at[slot, pl.ds(j * IDX_W, IDX_W)]],
            codes_v.at[slot, pl.ds(j * IDX_W, IDX_W)],
            sem_g,
        )

    def wait_codes(g, slot):
      for j in range(CHUNK // IDX_W):
        pltpu.make_async_copy(
            codes_hbm.at[tok_v.at[slot, pl.ds(j * IDX_W, IDX_W)]],
            codes_v.at[slot, pl.ds(j * IDX_W, IDX_W)],
            sem_g,
        ).wait()

    def out_copy_descriptor(g, slot):
      base = base0 + g * CHUNK
      return pltpu.make_async_copy(
          out_v.at[slot],
          out_hbm.at[pl.ds(base * N_CB, n_idx)],
          sem_w,
      )

    fetch(0, 0)

    def chunk_body(g, carry):
      p = g & 1
      q = 1 - p
      wait_codes(g, p)

      def ew(k, c):
        vals = plsc.load_gather(codes_v.at[p], [2 * k + row_off, col_idx])
        idx2_v[pl.ds(k * 16, 16)] = vals + pattern
        return c
      lax.fori_loop(0, n_idx // 16, ew, 0, unroll=8)

      @pl.when(g >= 2)
      def _():
        out_copy_descriptor(g - 2, p).wait()

      for j in range(n_idx // IDX_W):
        pltpu.async_copy(
            cb_sh.at[idx2_v.at[pl.ds(j * IDX_W, IDX_W)]],
            out_v.at[p, pl.ds(j * IDX_W, IDX_W)],
            sem_o,
        )

      @pl.when(g + 1 < n_chunks)
      def _():
        fetch(g + 1, q)

      for j in range(n_idx // IDX_W):
        pltpu.make_async_copy(
            cb_sh.at[idx2_v.at[pl.ds(j * IDX_W, IDX_W)]],
            out_v.at[p, pl.ds(j * IDX_W, IDX_W)],
            sem_o,
        ).wait()
      out_copy_descriptor(g, p).start()
      return carry

    lax.fori_loop(0, n_chunks, chunk_body, 0)

    out_copy_descriptor(n_chunks - 2, (n_chunks - 2) & 1).wait()
    out_copy_descriptor(n_chunks - 1, (n_chunks - 1) & 1).wait()

  return pq


def kernel(token_ids, codebooks, codes):
  b, l = token_ids.shape
  n_cb, n_codes, sub = codebooks.shape
  tok = token_ids.reshape(-1).astype(jnp.int32)
  cb = codebooks.reshape(n_cb * n_codes, sub).astype(jnp.float32)
  cds = codes.astype(jnp.int32)
  out = _build(b * l, codes.shape[0])(tok, cb, cds)
  return out.reshape(b, l, n_cb * sub)
